# Initial kernel scaffold; baseline (speedup 1.0000x reference)
import jax
import jax.numpy as jnp
from jax.experimental import pallas as pl


def kernel(vectorized_text, emb_table, W, b):
    raise NotImplementedError("write your pallas kernel here")


# speedup vs baseline: 1.0262x; 1.0262x over previous
import functools

import jax
import jax.numpy as jnp
from jax import lax
from jax.experimental import pallas as pl
from jax.experimental.pallas import tpu as pltpu
from jax.experimental.pallas import tpu_sc as plsc

B = 4096
L = 50
D = 64
O = 128
NC, NS = 2, 16
NW = NC * NS
BPW = B // NW


def _pool_body(vt_hbm, table_hbm, out_hbm, idx_v, acc_v, sem):
    wid = lax.axis_index("s") * NC + lax.axis_index("c")
    pltpu.sync_copy(vt_hbm.at[wid], idx_v)
    pltpu.async_copy(table_hbm.at[idx_v.at[0]], acc_v, sem).wait()

    def fire(j, carry):
        pltpu.async_copy(table_hbm.at[idx_v.at[j]], acc_v, sem, add=True)
        return carry

    lax.fori_loop(1, L, fire, 0)

    def drain(j, carry):
        pltpu.make_async_copy(table_hbm.at[idx_v.at[0]], acc_v, sem).wait()
        return carry

    lax.fori_loop(1, L, drain, 0)
    pltpu.sync_copy(acc_v, out_hbm.at[pl.ds(wid * BPW, BPW)])


_pool = functools.partial(
    pl.kernel,
    out_type=jax.ShapeDtypeStruct((B, D), jnp.float32),
    mesh=plsc.VectorSubcoreMesh(core_axis_name="c", subcore_axis_name="s"),
    scratch_types=[
        pltpu.VMEM((L, BPW), jnp.int32),
        pltpu.VMEM((BPW, D), jnp.float32),
        pltpu.SemaphoreType.DMA,
    ],
    compiler_params=pltpu.CompilerParams(use_tc_tiling_on_sc=False),
)(_pool_body)


def _linear_body(x_ref, w_ref, b_ref, o_ref):
    x = x_ref[...] * jnp.float32(1.0 / L)
    o_ref[...] = (
        lax.dot_general(
            x, w_ref[...], (((1,), (1,)), ((), ())),
            preferred_element_type=jnp.float32,
        )
        + b_ref[...]
    )


def _linear(pooled, W, b2d):
    blk = 512
    return pl.pallas_call(
        _linear_body,
        out_shape=jax.ShapeDtypeStruct((B, O), jnp.float32),
        grid=(B // blk,),
        in_specs=[
            pl.BlockSpec((blk, D), lambda i: (i, 0)),
            pl.BlockSpec((O, D), lambda i: (0, 0)),
            pl.BlockSpec((1, O), lambda i: (0, 0)),
        ],
        out_specs=pl.BlockSpec((blk, O), lambda i: (i, 0)),
    )(pooled, W, b2d)


def kernel(vectorized_text, emb_table, W, b):
    vt = (
        vectorized_text.astype(jnp.int32)
        .reshape(NW, BPW, L)
        .transpose(0, 2, 1)
    )
    pooled = _pool(vt, emb_table)
    return _linear(pooled, W, b.reshape(1, O))
